# baseline (speedup 1.0000x reference)
import functools

import jax
import jax.numpy as jnp
from jax import lax
from jax.experimental import pallas as pl
from jax.experimental.pallas import tpu as pltpu
from jax.experimental.pallas import tpu_sc as plsc

N_NODES = 10000
D_FEAT = 128
N_EDGES = 320000

NC = 2
NS = 16
NW = NC * NS
E_PER_TILE = N_EDGES // NW
K = 128
H = 39
E_HALF = H * K
REM = E_PER_TILE - 2 * E_HALF
IDX_BUF = 5120

ROWS_MAIN = 624
ROWS_TAIL_EXTRA = 16

NBLK = 5
RBLK = N_NODES // NBLK

def _scale_body(x_ref, n_ref, o_ref):
    o_ref[...] = x_ref[...] * n_ref[...]


def _combine_body(p0_ref, p1_ref, n_ref, o_ref):
    o_ref[...] = n_ref[...] * (p0_ref[...] + p1_ref[...])


_mesh = plsc.VectorSubcoreMesh(core_axis_name="c", subcore_axis_name="s")


@functools.partial(
    pl.kernel,
    out_type=(
        jax.ShapeDtypeStruct((N_NODES, D_FEAT), jnp.float32),
        jax.ShapeDtypeStruct((N_NODES, D_FEAT), jnp.float32),
    ),
    mesh=_mesh,
    scratch_types=[
        pltpu.VMEM((IDX_BUF,), jnp.int32),
        pltpu.VMEM((IDX_BUF,), jnp.int32),
        pltpu.VMEM((2, K, D_FEAT), jnp.float32),
        pltpu.VMEM((REM, D_FEAT), jnp.float32),
        pltpu.VMEM_SHARED((N_NODES, D_FEAT), jnp.float32),
        pltpu.SemaphoreType.DMA,
        pltpu.SemaphoreType.DMA,
        pltpu.SemaphoreType.DMA,
        pltpu.SemaphoreType.DMA,
        pltpu.SemaphoreType.DMA,
    ],
)
def _scatter_kernel(xn_hbm, ei_hbm, p0_hbm, p1_hbm,
                    src_v, dst_v, rows_v, rbuf, acc, g0, g1, g2, s0, s1):
    cid = lax.axis_index("c")
    sid = lax.axis_index("s")
    wid = cid * NS + sid

    base = wid * E_PER_TILE

    def _stage(half, sem_s, sem_d):
        bh = base + half * E_HALF
        head = lax.rem(bh, 128)
        astart = pl.multiple_of(bh - head, 128)
        ds = pltpu.async_copy(ei_hbm.at[0, pl.ds(astart, IDX_BUF)],
                              src_v, sem_s)
        dd = pltpu.async_copy(ei_hbm.at[1, pl.ds(astart, IDX_BUF)],
                              dst_v, sem_d)
        return head, ds, dd

    head0, i0s, i0d = _stage(0, g0, g1)

    zeros = jnp.zeros((16,), jnp.float32)

    def _zrow(i, carry):
        for c8 in range(D_FEAT // 16):
            rows_v[0, i, pl.ds(c8 * 16, 16)] = zeros
        return carry

    lax.fori_loop(0, K, _zrow, 0)

    zrow0 = sid * ROWS_MAIN
    zds = [pltpu.async_copy(rows_v.at[0],
                            acc.at[pl.ds(zrow0 + b * K, K)], s0)
           for b in range(ROWS_MAIN // K)]
    zds.append(pltpu.async_copy(
        rows_v.at[0, pl.ds(0, ROWS_MAIN % K)],
        acc.at[pl.ds(zrow0 + (ROWS_MAIN // K) * K, ROWS_MAIN % K)], s0))
    for d in zds:
        d.wait()

    @pl.when(sid == NS - 1)
    def _ztail():
        pltpu.sync_copy(rows_v.at[0, pl.ds(0, ROWS_TAIL_EXTRA)],
                        acc.at[pl.ds(NS * ROWS_MAIN, ROWS_TAIL_EXTRA)])

    i0s.wait()
    i0d.wait()
    plsc.subcore_barrier()

    buf0 = rows_v.at[0]
    buf1 = rows_v.at[1]

    for half in range(2):
        if half == 0:
            head = head0
        else:
            head, ihs, ihd = _stage(half, g0, g1)
            ihs.wait()
            ihd.wait()

        def _idx(ref, j):
            return ref.at[pl.ds(head + j * K, K)]

        def _gather(j, buf, sem):
            return pltpu.async_copy(xn_hbm.at[_idx(src_v, j)], buf, sem)

        def _gwait(j, buf, sem):
            pltpu.make_async_copy(xn_hbm.at[_idx(src_v, j)], buf,
                                  sem).wait()

        def _scatter(j, buf, sem):
            return pltpu.async_copy(buf, acc.at[_idx(dst_v, j)], sem,
                                    add=True)

        if half == 1:
            pltpu.async_copy(
                xn_hbm.at[src_v.at[pl.ds(head + H * K, REM)]], rbuf, g2)

        _gather(0, buf0, g0)

        def _pair(i, carry):
            j0 = 2 * i
            j1 = j0 + 1
            _gather(j1, buf1, g1)
            _gwait(j0, buf0, g0)
            _scatter(j0, buf0, s0).wait()
            _gather(j0 + 2, buf0, g0)
            _gwait(j1, buf1, g1)
            _scatter(j1, buf1, s1).wait()
            return carry

        lax.fori_loop(0, H // 2, _pair, 0)

        _gwait(H - 1, buf0, g0)
        _scatter(H - 1, buf0, s0).wait()

        if half == 1:
            ridx_d = dst_v.at[pl.ds(head + H * K, REM)]
            pltpu.make_async_copy(
                xn_hbm.at[src_v.at[pl.ds(head + H * K, REM)]],
                rbuf, g2).wait()
            pltpu.async_copy(rbuf, acc.at[ridx_d], s1, add=True).wait()

    plsc.subcore_barrier()

    row0 = sid * ROWS_MAIN
    tail0 = NS * ROWS_MAIN

    @pl.when(cid == 0)
    def _write0():
        pltpu.sync_copy(acc.at[pl.ds(row0, ROWS_MAIN)],
                        p0_hbm.at[pl.ds(row0, ROWS_MAIN)])

        @pl.when(sid == NS - 1)
        def _tail0():
            pltpu.sync_copy(acc.at[pl.ds(tail0, ROWS_TAIL_EXTRA)],
                            p0_hbm.at[pl.ds(tail0, ROWS_TAIL_EXTRA)])

    @pl.when(cid == 1)
    def _write1():
        pltpu.sync_copy(acc.at[pl.ds(row0, ROWS_MAIN)],
                        p1_hbm.at[pl.ds(row0, ROWS_MAIN)])

        @pl.when(sid == NS - 1)
        def _tail1():
            pltpu.sync_copy(acc.at[pl.ds(tail0, ROWS_TAIL_EXTRA)],
                            p1_hbm.at[pl.ds(tail0, ROWS_TAIL_EXTRA)])


def kernel(x, norm, edge_index):
    if edge_index.dtype == jnp.int32:
        ei = edge_index
    else:
        ei = edge_index.astype(jnp.int32)

    xn = pl.pallas_call(
        _scale_body,
        grid=(NBLK,),
        in_specs=[
            pl.BlockSpec((RBLK, D_FEAT), lambda i: (i, 0)),
            pl.BlockSpec((RBLK, 1), lambda i: (i, 0)),
        ],
        out_specs=pl.BlockSpec((RBLK, D_FEAT), lambda i: (i, 0)),
        out_shape=jax.ShapeDtypeStruct((N_NODES, D_FEAT), jnp.float32),
    )(x, norm)

    p0, p1 = _scatter_kernel(xn, ei)

    y = pl.pallas_call(
        _combine_body,
        grid=(NBLK,),
        in_specs=[
            pl.BlockSpec((RBLK, D_FEAT), lambda i: (i, 0)),
            pl.BlockSpec((RBLK, D_FEAT), lambda i: (i, 0)),
            pl.BlockSpec((RBLK, 1), lambda i: (i, 0)),
        ],
        out_specs=pl.BlockSpec((RBLK, D_FEAT), lambda i: (i, 0)),
        out_shape=jax.ShapeDtypeStruct((N_NODES, D_FEAT), jnp.float32),
    )(p0, p1, norm)
    return y


# speedup vs baseline: 1.0046x; 1.0046x over previous
import functools

import jax
import jax.numpy as jnp
from jax import lax
from jax.experimental import pallas as pl
from jax.experimental.pallas import tpu as pltpu
from jax.experimental.pallas import tpu_sc as plsc

N_NODES = 10000
D_FEAT = 128
N_EDGES = 320000

NC = 2
NS = 16
NW = NC * NS
E_PER_TILE = N_EDGES // NW
K = 128
H = 39
E_HALF = H * K
REM = E_PER_TILE - 2 * E_HALF
IDX_BUF = 5120

ROWS_MAIN = 624
ROWS_TAIL_EXTRA = 16

NBLK = 5
RBLK = N_NODES // NBLK

def _scale_body(x_ref, n_ref, o_ref):
    o_ref[...] = x_ref[...] * n_ref[...]


def _combine_body(p0_ref, p1_ref, n_ref, o_ref):
    o_ref[...] = n_ref[...] * (p0_ref[...] + p1_ref[...])


_mesh = plsc.VectorSubcoreMesh(core_axis_name="c", subcore_axis_name="s")


@functools.partial(
    pl.kernel,
    out_type=(
        jax.ShapeDtypeStruct((N_NODES, D_FEAT), jnp.float32),
        jax.ShapeDtypeStruct((N_NODES, D_FEAT), jnp.float32),
    ),
    mesh=_mesh,
    scratch_types=[
        pltpu.VMEM((IDX_BUF,), jnp.int32),
        pltpu.VMEM((IDX_BUF,), jnp.int32),
        pltpu.VMEM((2, K, D_FEAT), jnp.float32),
        pltpu.VMEM((REM, D_FEAT), jnp.float32),
        pltpu.VMEM_SHARED((N_NODES, D_FEAT), jnp.float32),
        pltpu.SemaphoreType.DMA,
        pltpu.SemaphoreType.DMA,
        pltpu.SemaphoreType.DMA,
        pltpu.SemaphoreType.DMA,
        pltpu.SemaphoreType.DMA,
    ],
)
def _scatter_kernel(xn_hbm, ei_hbm, p0_hbm, p1_hbm,
                    src_v, dst_v, rows_v, rbuf, acc, g0, g1, g2, s0, s1):
    cid = lax.axis_index("c")
    sid = lax.axis_index("s")
    wid = cid * NS + sid

    base = wid * E_PER_TILE

    def _stage(half, sem_s, sem_d):
        bh = base + half * E_HALF
        head = lax.rem(bh, 128)
        astart = pl.multiple_of(bh - head, 128)
        ds = pltpu.async_copy(ei_hbm.at[0, pl.ds(astart, IDX_BUF)],
                              src_v, sem_s)
        dd = pltpu.async_copy(ei_hbm.at[1, pl.ds(astart, IDX_BUF)],
                              dst_v, sem_d)
        return head, ds, dd

    head0, i0s, i0d = _stage(0, g0, g1)

    zeros = jnp.zeros((16,), jnp.float32)

    def _zrow(i, carry):
        for c8 in range(D_FEAT // 16):
            rows_v[0, i, pl.ds(c8 * 16, 16)] = zeros
        return carry

    lax.fori_loop(0, K, _zrow, 0)

    zrow0 = sid * ROWS_MAIN
    zds = [pltpu.async_copy(rows_v.at[0],
                            acc.at[pl.ds(zrow0 + b * K, K)], s0)
           for b in range(ROWS_MAIN // K)]
    zds.append(pltpu.async_copy(
        rows_v.at[0, pl.ds(0, ROWS_MAIN % K)],
        acc.at[pl.ds(zrow0 + (ROWS_MAIN // K) * K, ROWS_MAIN % K)], s0))
    for d in zds:
        d.wait()

    @pl.when(sid == NS - 1)
    def _ztail():
        pltpu.sync_copy(rows_v.at[0, pl.ds(0, ROWS_TAIL_EXTRA)],
                        acc.at[pl.ds(NS * ROWS_MAIN, ROWS_TAIL_EXTRA)])

    i0s.wait()
    i0d.wait()
    plsc.subcore_barrier()

    buf0 = rows_v.at[0]
    buf1 = rows_v.at[1]

    for half in range(2):
        if half == 0:
            head = head0
        else:
            head, ihs, ihd = _stage(half, g0, g1)
            ihs.wait()
            ihd.wait()

        def _idx(ref, j):
            return ref.at[pl.ds(head + j * K, K)]

        def _gather(j, buf, sem):
            return pltpu.async_copy(xn_hbm.at[_idx(src_v, j)], buf, sem)

        def _gwait(j, buf, sem):
            pltpu.make_async_copy(xn_hbm.at[_idx(src_v, j)], buf,
                                  sem).wait()

        def _scatter(j, buf, sem):
            return pltpu.async_copy(buf, acc.at[_idx(dst_v, j)], sem,
                                    add=True)

        if half == 1:
            pltpu.async_copy(
                xn_hbm.at[src_v.at[pl.ds(head + H * K, REM)]], rbuf, g2)

        _gather(0, buf0, g0)

        def _pair(i, carry):
            j0 = 2 * i
            j1 = j0 + 1
            _gather(j1, buf1, g1)
            _gwait(j0, buf0, g0)
            _scatter(j0, buf0, s0).wait()
            _gather(j0 + 2, buf0, g0)
            _gwait(j1, buf1, g1)
            _scatter(j1, buf1, s1).wait()
            return carry

        lax.fori_loop(0, H // 2, _pair, 0)

        _gwait(H - 1, buf0, g0)
        _scatter(H - 1, buf0, s0).wait()

        if half == 1:
            ridx_d = dst_v.at[pl.ds(head + H * K, REM)]
            pltpu.make_async_copy(
                xn_hbm.at[src_v.at[pl.ds(head + H * K, REM)]],
                rbuf, g2).wait()
            pltpu.async_copy(rbuf, acc.at[ridx_d], s1, add=True).wait()

    plsc.subcore_barrier()

    row0 = sid * ROWS_MAIN
    tail0 = NS * ROWS_MAIN

    @pl.when(cid == 0)
    def _write0():
        pltpu.sync_copy(acc.at[pl.ds(row0, ROWS_MAIN)],
                        p0_hbm.at[pl.ds(row0, ROWS_MAIN)])

        @pl.when(sid == NS - 1)
        def _tail0():
            pltpu.sync_copy(acc.at[pl.ds(tail0, ROWS_TAIL_EXTRA)],
                            p0_hbm.at[pl.ds(tail0, ROWS_TAIL_EXTRA)])

    @pl.when(cid == 1)
    def _write1():
        pltpu.sync_copy(acc.at[pl.ds(row0, ROWS_MAIN)],
                        p1_hbm.at[pl.ds(row0, ROWS_MAIN)])

        @pl.when(sid == NS - 1)
        def _tail1():
            pltpu.sync_copy(acc.at[pl.ds(tail0, ROWS_TAIL_EXTRA)],
                            p1_hbm.at[pl.ds(tail0, ROWS_TAIL_EXTRA)])


def kernel(x, norm, edge_index):
    if edge_index.dtype == jnp.int32:
        ei = edge_index
    else:
        ei = edge_index.astype(jnp.int32)

    xn = pl.pallas_call(
        _scale_body,
        out_shape=jax.ShapeDtypeStruct((N_NODES, D_FEAT), jnp.float32),
    )(x, norm)

    p0, p1 = _scatter_kernel(xn, ei)

    y = pl.pallas_call(
        _combine_body,
        out_shape=jax.ShapeDtypeStruct((N_NODES, D_FEAT), jnp.float32),
    )(p0, p1, norm)
    return y


# speedup vs baseline: 1.0281x; 1.0234x over previous
import functools

import jax
import jax.numpy as jnp
from jax import lax
from jax.experimental import pallas as pl
from jax.experimental.pallas import tpu as pltpu
from jax.experimental.pallas import tpu_sc as plsc

N_NODES = 10000
D_FEAT = 128
N_EDGES = 320000

NC = 2
NS = 16
NW = NC * NS
E_PER_TILE = N_EDGES // NW
K = 128
H = 39
E_HALF = H * K
REM = E_PER_TILE - 2 * E_HALF
IDX_BUF = 5120

ROWS_MAIN = 624
ROWS_TAIL_EXTRA = 16

NBLK = 5
RBLK = N_NODES // NBLK

def _scale_body(x_ref, n_ref, o_ref):
    o_ref[...] = x_ref[...] * n_ref[...].reshape(N_NODES, 1)


def _combine_body(p0_ref, p1_ref, n_ref, o_ref):
    o_ref[...] = n_ref[...] * (p0_ref[...] + p1_ref[...])


_mesh = plsc.VectorSubcoreMesh(core_axis_name="c", subcore_axis_name="s")


@functools.partial(
    pl.kernel,
    out_type=(
        jax.ShapeDtypeStruct((N_NODES, D_FEAT), jnp.float32),
        jax.ShapeDtypeStruct((N_NODES, D_FEAT), jnp.float32),
    ),
    mesh=_mesh,
    scratch_types=[
        pltpu.VMEM((IDX_BUF,), jnp.int32),
        pltpu.VMEM((IDX_BUF,), jnp.int32),
        pltpu.VMEM((2, K, D_FEAT), jnp.float32),
        pltpu.VMEM((REM, D_FEAT), jnp.float32),
        pltpu.VMEM_SHARED((N_NODES, D_FEAT), jnp.float32),
        pltpu.SemaphoreType.DMA,
        pltpu.SemaphoreType.DMA,
        pltpu.SemaphoreType.DMA,
        pltpu.SemaphoreType.DMA,
        pltpu.SemaphoreType.DMA,
    ],
)
def _scatter_kernel(xn_hbm, ei_hbm, p0_hbm, p1_hbm,
                    src_v, dst_v, rows_v, rbuf, acc, g0, g1, g2, s0, s1):
    cid = lax.axis_index("c")
    sid = lax.axis_index("s")
    wid = cid * NS + sid

    base = wid * E_PER_TILE

    def _stage(half, sem_s, sem_d):
        bh = base + half * E_HALF
        head = lax.rem(bh, 128)
        astart = pl.multiple_of(bh - head, 128)
        ds = pltpu.async_copy(ei_hbm.at[0, pl.ds(astart, IDX_BUF)],
                              src_v, sem_s)
        dd = pltpu.async_copy(ei_hbm.at[1, pl.ds(astart, IDX_BUF)],
                              dst_v, sem_d)
        return head, ds, dd

    head0, i0s, i0d = _stage(0, g0, g1)

    zeros = jnp.zeros((16,), jnp.float32)

    def _zrow(i, carry):
        for c8 in range(D_FEAT // 16):
            rows_v[0, i, pl.ds(c8 * 16, 16)] = zeros
        return carry

    lax.fori_loop(0, K, _zrow, 0)

    zrow0 = sid * ROWS_MAIN
    zds = [pltpu.async_copy(rows_v.at[0],
                            acc.at[pl.ds(zrow0 + b * K, K)], s0)
           for b in range(ROWS_MAIN // K)]
    zds.append(pltpu.async_copy(
        rows_v.at[0, pl.ds(0, ROWS_MAIN % K)],
        acc.at[pl.ds(zrow0 + (ROWS_MAIN // K) * K, ROWS_MAIN % K)], s0))
    for d in zds:
        d.wait()

    @pl.when(sid == NS - 1)
    def _ztail():
        pltpu.sync_copy(rows_v.at[0, pl.ds(0, ROWS_TAIL_EXTRA)],
                        acc.at[pl.ds(NS * ROWS_MAIN, ROWS_TAIL_EXTRA)])

    i0s.wait()
    i0d.wait()
    plsc.subcore_barrier()

    buf0 = rows_v.at[0]
    buf1 = rows_v.at[1]

    for half in range(2):
        if half == 0:
            head = head0
        else:
            head, ihs, ihd = _stage(half, g0, g1)
            ihs.wait()
            ihd.wait()

        def _idx(ref, j):
            return ref.at[pl.ds(head + j * K, K)]

        def _gather(j, buf, sem):
            return pltpu.async_copy(xn_hbm.at[_idx(src_v, j)], buf, sem)

        def _gwait(j, buf, sem):
            pltpu.make_async_copy(xn_hbm.at[_idx(src_v, j)], buf,
                                  sem).wait()

        def _scatter(j, buf, sem):
            return pltpu.async_copy(buf, acc.at[_idx(dst_v, j)], sem,
                                    add=True)

        if half == 1:
            pltpu.async_copy(
                xn_hbm.at[src_v.at[pl.ds(head + H * K, REM)]], rbuf, g2)

        _gather(0, buf0, g0)

        def _pair(i, carry):
            j0 = 2 * i
            j1 = j0 + 1
            _gather(j1, buf1, g1)
            _gwait(j0, buf0, g0)
            _scatter(j0, buf0, s0).wait()
            _gather(j0 + 2, buf0, g0)
            _gwait(j1, buf1, g1)
            _scatter(j1, buf1, s1).wait()
            return carry

        lax.fori_loop(0, H // 2, _pair, 0)

        _gwait(H - 1, buf0, g0)
        _scatter(H - 1, buf0, s0).wait()

        if half == 1:
            ridx_d = dst_v.at[pl.ds(head + H * K, REM)]
            pltpu.make_async_copy(
                xn_hbm.at[src_v.at[pl.ds(head + H * K, REM)]],
                rbuf, g2).wait()
            pltpu.async_copy(rbuf, acc.at[ridx_d], s1, add=True).wait()

    plsc.subcore_barrier()

    row0 = sid * ROWS_MAIN
    tail0 = NS * ROWS_MAIN

    @pl.when(cid == 0)
    def _write0():
        pltpu.sync_copy(acc.at[pl.ds(row0, ROWS_MAIN)],
                        p0_hbm.at[pl.ds(row0, ROWS_MAIN)])

        @pl.when(sid == NS - 1)
        def _tail0():
            pltpu.sync_copy(acc.at[pl.ds(tail0, ROWS_TAIL_EXTRA)],
                            p0_hbm.at[pl.ds(tail0, ROWS_TAIL_EXTRA)])

    @pl.when(cid == 1)
    def _write1():
        pltpu.sync_copy(acc.at[pl.ds(row0, ROWS_MAIN)],
                        p1_hbm.at[pl.ds(row0, ROWS_MAIN)])

        @pl.when(sid == NS - 1)
        def _tail1():
            pltpu.sync_copy(acc.at[pl.ds(tail0, ROWS_TAIL_EXTRA)],
                            p1_hbm.at[pl.ds(tail0, ROWS_TAIL_EXTRA)])


def kernel(x, norm, edge_index):
    if edge_index.dtype == jnp.int32:
        ei = edge_index
    else:
        ei = edge_index.astype(jnp.int32)

    xn = pl.pallas_call(
        _scale_body,
        out_shape=jax.ShapeDtypeStruct((N_NODES, D_FEAT), jnp.float32),
    )(x, norm.reshape(N_NODES))

    p0, p1 = _scatter_kernel(xn, ei)

    y = pl.pallas_call(
        _combine_body,
        out_shape=jax.ShapeDtypeStruct((N_NODES, D_FEAT), jnp.float32),
    )(p0, p1, norm)
    return y


# speedup vs baseline: 1.0409x; 1.0125x over previous
import functools

import jax
import jax.numpy as jnp
from jax import lax
from jax.experimental import pallas as pl
from jax.experimental.pallas import tpu as pltpu
from jax.experimental.pallas import tpu_sc as plsc

N_NODES = 10000
D_FEAT = 128
N_EDGES = 320000

NC = 2
NS = 16
NW = NC * NS
E_PER_TILE = N_EDGES // NW
K = 128
H = 39
E_HALF = H * K
REM = E_PER_TILE - 2 * E_HALF
IDX_BUF = 5120

ROWS_MAIN = 624
ROWS_TAIL_EXTRA = 16

NBLK = 5
RBLK = N_NODES // NBLK

def _scale_body(x_ref, n_ref, o_ref):
    o_ref[...] = x_ref[...] * n_ref[...].reshape(N_NODES, 1)


def _combine_body(p0_ref, p1_ref, n_ref, o_ref):
    o_ref[...] = n_ref[...].reshape(N_NODES, 1) * (p0_ref[...] + p1_ref[...])


_mesh = plsc.VectorSubcoreMesh(core_axis_name="c", subcore_axis_name="s")


@functools.partial(
    pl.kernel,
    out_type=(
        jax.ShapeDtypeStruct((N_NODES, D_FEAT), jnp.float32),
        jax.ShapeDtypeStruct((N_NODES, D_FEAT), jnp.float32),
    ),
    mesh=_mesh,
    scratch_types=[
        pltpu.VMEM((IDX_BUF,), jnp.int32),
        pltpu.VMEM((IDX_BUF,), jnp.int32),
        pltpu.VMEM((2, K, D_FEAT), jnp.float32),
        pltpu.VMEM((REM, D_FEAT), jnp.float32),
        pltpu.VMEM_SHARED((N_NODES, D_FEAT), jnp.float32),
        pltpu.SemaphoreType.DMA,
        pltpu.SemaphoreType.DMA,
        pltpu.SemaphoreType.DMA,
        pltpu.SemaphoreType.DMA,
        pltpu.SemaphoreType.DMA,
    ],
)
def _scatter_kernel(xn_hbm, ei_hbm, p0_hbm, p1_hbm,
                    src_v, dst_v, rows_v, rbuf, acc, g0, g1, g2, s0, s1):
    cid = lax.axis_index("c")
    sid = lax.axis_index("s")
    wid = cid * NS + sid

    base = wid * E_PER_TILE

    def _stage(half, sem_s, sem_d):
        bh = base + half * E_HALF
        head = lax.rem(bh, 128)
        astart = pl.multiple_of(bh - head, 128)
        ds = pltpu.async_copy(ei_hbm.at[0, pl.ds(astart, IDX_BUF)],
                              src_v, sem_s)
        dd = pltpu.async_copy(ei_hbm.at[1, pl.ds(astart, IDX_BUF)],
                              dst_v, sem_d)
        return head, ds, dd

    head0, i0s, i0d = _stage(0, g0, g1)

    zeros = jnp.zeros((16,), jnp.float32)

    def _zrow(i, carry):
        for c8 in range(D_FEAT // 16):
            rows_v[0, i, pl.ds(c8 * 16, 16)] = zeros
        return carry

    lax.fori_loop(0, K, _zrow, 0)

    zrow0 = sid * ROWS_MAIN
    zds = [pltpu.async_copy(rows_v.at[0],
                            acc.at[pl.ds(zrow0 + b * K, K)], s0)
           for b in range(ROWS_MAIN // K)]
    zds.append(pltpu.async_copy(
        rows_v.at[0, pl.ds(0, ROWS_MAIN % K)],
        acc.at[pl.ds(zrow0 + (ROWS_MAIN // K) * K, ROWS_MAIN % K)], s0))
    for d in zds:
        d.wait()

    @pl.when(sid == NS - 1)
    def _ztail():
        pltpu.sync_copy(rows_v.at[0, pl.ds(0, ROWS_TAIL_EXTRA)],
                        acc.at[pl.ds(NS * ROWS_MAIN, ROWS_TAIL_EXTRA)])

    i0s.wait()
    i0d.wait()
    plsc.subcore_barrier()

    buf0 = rows_v.at[0]
    buf1 = rows_v.at[1]

    for half in range(2):
        if half == 0:
            head = head0
        else:
            head, ihs, ihd = _stage(half, g0, g1)
            ihs.wait()
            ihd.wait()

        def _idx(ref, j):
            return ref.at[pl.ds(head + j * K, K)]

        def _gather(j, buf, sem):
            return pltpu.async_copy(xn_hbm.at[_idx(src_v, j)], buf, sem)

        def _gwait(j, buf, sem):
            pltpu.make_async_copy(xn_hbm.at[_idx(src_v, j)], buf,
                                  sem).wait()

        def _scatter(j, buf, sem):
            return pltpu.async_copy(buf, acc.at[_idx(dst_v, j)], sem,
                                    add=True)

        if half == 1:
            pltpu.async_copy(
                xn_hbm.at[src_v.at[pl.ds(head + H * K, REM)]], rbuf, g2)

        _gather(0, buf0, g0)

        def _pair(i, carry):
            j0 = 2 * i
            j1 = j0 + 1
            _gather(j1, buf1, g1)
            _gwait(j0, buf0, g0)
            _scatter(j0, buf0, s0).wait()
            _gather(j0 + 2, buf0, g0)
            _gwait(j1, buf1, g1)
            _scatter(j1, buf1, s1).wait()
            return carry

        lax.fori_loop(0, H // 2, _pair, 0)

        _gwait(H - 1, buf0, g0)
        _scatter(H - 1, buf0, s0).wait()

        if half == 1:
            ridx_d = dst_v.at[pl.ds(head + H * K, REM)]
            pltpu.make_async_copy(
                xn_hbm.at[src_v.at[pl.ds(head + H * K, REM)]],
                rbuf, g2).wait()
            pltpu.async_copy(rbuf, acc.at[ridx_d], s1, add=True).wait()

    plsc.subcore_barrier()

    row0 = sid * ROWS_MAIN
    tail0 = NS * ROWS_MAIN

    @pl.when(cid == 0)
    def _write0():
        pltpu.sync_copy(acc.at[pl.ds(row0, ROWS_MAIN)],
                        p0_hbm.at[pl.ds(row0, ROWS_MAIN)])

        @pl.when(sid == NS - 1)
        def _tail0():
            pltpu.sync_copy(acc.at[pl.ds(tail0, ROWS_TAIL_EXTRA)],
                            p0_hbm.at[pl.ds(tail0, ROWS_TAIL_EXTRA)])

    @pl.when(cid == 1)
    def _write1():
        pltpu.sync_copy(acc.at[pl.ds(row0, ROWS_MAIN)],
                        p1_hbm.at[pl.ds(row0, ROWS_MAIN)])

        @pl.when(sid == NS - 1)
        def _tail1():
            pltpu.sync_copy(acc.at[pl.ds(tail0, ROWS_TAIL_EXTRA)],
                            p1_hbm.at[pl.ds(tail0, ROWS_TAIL_EXTRA)])


def kernel(x, norm, edge_index):
    if edge_index.dtype == jnp.int32:
        ei = edge_index
    else:
        ei = edge_index.astype(jnp.int32)

    xn = pl.pallas_call(
        _scale_body,
        out_shape=jax.ShapeDtypeStruct((N_NODES, D_FEAT), jnp.float32),
    )(x, norm.reshape(N_NODES))

    p0, p1 = _scatter_kernel(xn, ei)

    y = pl.pallas_call(
        _combine_body,
        out_shape=jax.ShapeDtypeStruct((N_NODES, D_FEAT), jnp.float32),
    )(p0, p1, norm.reshape(N_NODES))
    return y


# speedup vs baseline: 1.0557x; 1.0142x over previous
import functools

import jax
import jax.numpy as jnp
from jax import lax
from jax.experimental import pallas as pl
from jax.experimental.pallas import tpu as pltpu
from jax.experimental.pallas import tpu_sc as plsc

N_NODES = 10000
D_FEAT = 128
N_EDGES = 320000

NC = 2
NS = 16
NW = NC * NS
E_PER_TILE = N_EDGES // NW
K = 128
H = 39
E_HALF = H * K
REM = E_PER_TILE - 2 * E_HALF
IDX_BUF = 5120

ROWS_MAIN = 624
ROWS_TAIL_EXTRA = 16

NBLK = 5
RBLK = N_NODES // NBLK

def _scale_body(x_ref, n_ref, o_ref):
    o_ref[...] = x_ref[...] * n_ref[...].reshape(N_NODES, 1)


def _combine_body(p0_ref, p1_ref, n_ref, o_ref):
    o_ref[...] = n_ref[...].reshape(N_NODES, 1) * (p0_ref[...] + p1_ref[...])


_mesh = plsc.VectorSubcoreMesh(core_axis_name="c", subcore_axis_name="s")


@functools.partial(
    pl.kernel,
    out_type=(
        jax.ShapeDtypeStruct((N_NODES, D_FEAT), jnp.float32),
        jax.ShapeDtypeStruct((N_NODES, D_FEAT), jnp.float32),
    ),
    mesh=_mesh,
    scratch_types=[
        pltpu.VMEM((IDX_BUF,), jnp.int32),
        pltpu.VMEM((IDX_BUF,), jnp.int32),
        pltpu.VMEM((2, K, D_FEAT), jnp.float32),
        pltpu.VMEM((REM, D_FEAT), jnp.float32),
        pltpu.VMEM_SHARED((N_NODES, D_FEAT), jnp.float32),
        pltpu.SemaphoreType.DMA,
        pltpu.SemaphoreType.DMA,
        pltpu.SemaphoreType.DMA,
        pltpu.SemaphoreType.DMA,
        pltpu.SemaphoreType.DMA,
    ],
)
def _scatter_kernel(xn_hbm, ei_hbm, p0_hbm, p1_hbm,
                    src_v, dst_v, rows_v, rbuf, acc, g0, g1, g2, s0, s1):
    cid = lax.axis_index("c")
    sid = lax.axis_index("s")
    wid = cid * NS + sid

    base = wid * E_PER_TILE

    def _stage(half, sem_s, sem_d):
        bh = base + half * E_HALF
        head = lax.rem(bh, 128)
        astart = pl.multiple_of(bh - head, 128)
        ds = pltpu.async_copy(ei_hbm.at[0, pl.ds(astart, IDX_BUF)],
                              src_v, sem_s)
        dd = pltpu.async_copy(ei_hbm.at[1, pl.ds(astart, IDX_BUF)],
                              dst_v, sem_d)
        return head, ds, dd

    head0, i0s, i0d = _stage(0, g0, g1)

    zeros = jnp.zeros((16,), jnp.float32)

    def _zrow(i, carry):
        for c8 in range(D_FEAT // 16):
            rows_v[0, i, pl.ds(c8 * 16, 16)] = zeros
        return carry

    lax.fori_loop(0, K, _zrow, 0)

    zrow0 = sid * ROWS_MAIN
    zds = [pltpu.async_copy(rows_v.at[0],
                            acc.at[pl.ds(zrow0 + b * K, K)], s0)
           for b in range(ROWS_MAIN // K)]
    zds.append(pltpu.async_copy(
        rows_v.at[0, pl.ds(0, ROWS_MAIN % K)],
        acc.at[pl.ds(zrow0 + (ROWS_MAIN // K) * K, ROWS_MAIN % K)], s0))

    i0s.wait()
    i0d.wait()

    buf0 = rows_v.at[0]
    buf1 = rows_v.at[1]

    for half in range(2):
        if half == 0:
            head = head0
        else:
            head, ihs, ihd = _stage(half, g0, g1)
            ihs.wait()
            ihd.wait()

        def _idx(ref, j):
            return ref.at[pl.ds(head + j * K, K)]

        def _gather(j, buf, sem):
            return pltpu.async_copy(xn_hbm.at[_idx(src_v, j)], buf, sem)

        def _gwait(j, buf, sem):
            pltpu.make_async_copy(xn_hbm.at[_idx(src_v, j)], buf,
                                  sem).wait()

        def _scatter(j, buf, sem):
            return pltpu.async_copy(buf, acc.at[_idx(dst_v, j)], sem,
                                    add=True)

        if half == 1:
            pltpu.async_copy(
                xn_hbm.at[src_v.at[pl.ds(head + H * K, REM)]], rbuf, g2)

        _gather(0, buf1, g1)

        if half == 0:
            for d in zds:
                d.wait()

            @pl.when(sid == NS - 1)
            def _ztail():
                pltpu.sync_copy(
                    rows_v.at[0, pl.ds(0, ROWS_TAIL_EXTRA)],
                    acc.at[pl.ds(NS * ROWS_MAIN, ROWS_TAIL_EXTRA)])

            plsc.subcore_barrier()

        def _pair(i, carry):
            j0 = 2 * i
            j1 = j0 + 1
            _gather(j1, buf0, g0)
            _gwait(j0, buf1, g1)
            _scatter(j0, buf1, s1).wait()
            _gather(j0 + 2, buf1, g1)
            _gwait(j1, buf0, g0)
            _scatter(j1, buf0, s0).wait()
            return carry

        lax.fori_loop(0, H // 2, _pair, 0)

        _gwait(H - 1, buf1, g1)
        _scatter(H - 1, buf1, s1).wait()

        if half == 1:
            ridx_d = dst_v.at[pl.ds(head + H * K, REM)]
            pltpu.make_async_copy(
                xn_hbm.at[src_v.at[pl.ds(head + H * K, REM)]],
                rbuf, g2).wait()
            pltpu.async_copy(rbuf, acc.at[ridx_d], s1, add=True).wait()

    plsc.subcore_barrier()

    row0 = sid * ROWS_MAIN
    tail0 = NS * ROWS_MAIN

    @pl.when(cid == 0)
    def _write0():
        pltpu.sync_copy(acc.at[pl.ds(row0, ROWS_MAIN)],
                        p0_hbm.at[pl.ds(row0, ROWS_MAIN)])

        @pl.when(sid == NS - 1)
        def _tail0():
            pltpu.sync_copy(acc.at[pl.ds(tail0, ROWS_TAIL_EXTRA)],
                            p0_hbm.at[pl.ds(tail0, ROWS_TAIL_EXTRA)])

    @pl.when(cid == 1)
    def _write1():
        pltpu.sync_copy(acc.at[pl.ds(row0, ROWS_MAIN)],
                        p1_hbm.at[pl.ds(row0, ROWS_MAIN)])

        @pl.when(sid == NS - 1)
        def _tail1():
            pltpu.sync_copy(acc.at[pl.ds(tail0, ROWS_TAIL_EXTRA)],
                            p1_hbm.at[pl.ds(tail0, ROWS_TAIL_EXTRA)])


def kernel(x, norm, edge_index):
    if edge_index.dtype == jnp.int32:
        ei = edge_index
    else:
        ei = edge_index.astype(jnp.int32)

    xn = pl.pallas_call(
        _scale_body,
        out_shape=jax.ShapeDtypeStruct((N_NODES, D_FEAT), jnp.float32),
    )(x, norm.reshape(N_NODES))

    p0, p1 = _scatter_kernel(xn, ei)

    y = pl.pallas_call(
        _combine_body,
        out_shape=jax.ShapeDtypeStruct((N_NODES, D_FEAT), jnp.float32),
    )(p0, p1, norm.reshape(N_NODES))
    return y
